# baseline (speedup 1.0000x reference)
import functools

import jax
import jax.numpy as jnp
from jax import lax
from jax.experimental import pallas as pl
from jax.experimental.pallas import tpu as pltpu
from jax.experimental.pallas import tpu_sc as plsc

_NC, _NS, _L = 2, 16, 16
_NW = _NC * _NS


def _seg_sum_kernel(n, d, e, with_gather, dtype=jnp.float32):
    per_w = e // _NW
    ch = 80
    n_ch = per_w // ch
    NB = 2
    n_grp = n_ch // NB
    n_tail = n_ch - n_grp * NB
    rpt = (n // _NS) // 8 * 8
    last_extra = n - _NS * rpt
    nz = rpt // ch
    zrem = rpt - nz * ch
    mesh = plsc.VectorSubcoreMesh(core_axis_name="c", subcore_axis_name="s")

    scratch = [
        pltpu.VMEM((NB, ch), jnp.int32),
        pltpu.VMEM((n_ch, ch), jnp.int32),
        pltpu.VMEM((NB, ch, d), dtype),
        pltpu.VMEM_SHARED((n, d), dtype),
    ] + [pltpu.SemaphoreType.DMA] * 8

    @functools.partial(
        pl.kernel,
        out_type=jax.ShapeDtypeStruct((_NC * n, d), dtype),
        mesh=mesh,
        scratch_types=scratch,
        compiler_params=pltpu.CompilerParams(use_tc_tiling_on_sc=False),
    )
    def seg(x_hbm, src_hbm, dst_hbm, out_hbm, sidx, didx, rows, acc,
            *sems):
        gsem, isem, ssem = sems[:NB], sems[NB:2 * NB], sems[2 * NB:3 * NB]
        cid = lax.axis_index("c")
        sid = lax.axis_index("s")
        wid = cid * _NS + sid
        base = wid * per_w

        def _wait_idx(sem):
            pltpu.make_async_copy(dst_hbm.at[pl.ds(0, ch)], sidx.at[0],
                                  sem).wait()

        def _wait_gather(b):
            pltpu.make_async_copy(x_hbm.at[sidx.at[b]], rows.at[b],
                                  gsem[b]).wait()

        ni = len(sems)
        ip_tail = n_ch % ni

        def _idx(t, _):
            for b in range(ni):
                k = t * ni + b

                @pl.when(t > 0)
                def _():
                    _wait_idx(sems[b])
                pltpu.async_copy(dst_hbm.at[pl.ds(base + k * ch, ch)],
                                 didx.at[k], sems[b])
            return 0
        lax.fori_loop(0, n_ch // ni, _idx, 0)
        for r in range(ip_tail):
            k = (n_ch // ni) * ni + r
            _wait_idx(sems[r])
            pltpu.async_copy(dst_hbm.at[pl.ds(base + k * ch, ch)],
                             didx.at[k], sems[r])

        lanes = _L * 4 // jnp.dtype(dtype).itemsize

        def _zr(r, _):
            def _zc(c, _):
                rows[0, r, pl.ds(c * lanes, lanes)] = jnp.zeros((lanes,),
                                                                dtype)
                return 0
            return lax.fori_loop(0, d // lanes, _zc, 0)
        lax.fori_loop(0, ch, _zr, 0)

        for b in range(ni):
            _wait_idx(sems[b])

        row0 = pl.multiple_of(sid * rpt, 8)
        for j in range(nz):
            pltpu.sync_copy(rows.at[0], acc.at[pl.ds(row0 + j * ch, ch)])
        pltpu.sync_copy(rows.at[0, pl.ds(0, zrem)],
                        acc.at[pl.ds(row0 + nz * ch, zrem)])

        @pl.when(sid == _NS - 1)
        def _():
            pltpu.sync_copy(rows.at[0, pl.ds(0, last_extra)],
                            acc.at[pl.ds(_NS * rpt, last_extra)])

        if with_gather:
            for b in range(NB):
                pltpu.async_copy(src_hbm.at[pl.ds(base + b * ch, ch)],
                                 sidx.at[b], isem[b])
            _wait_idx(isem[0])
            pltpu.async_copy(x_hbm.at[sidx.at[0]], rows.at[0], gsem[0])
        else:
            def _or(r, _):
                def _oc(c, _):
                    rows[0, r, pl.ds(c * lanes, lanes)] = jnp.full(
                        (lanes,), 1.0, dtype)
                    return 0
                return lax.fori_loop(0, d // lanes, _oc, 0)
            lax.fori_loop(0, ch, _or, 0)
        plsc.subcore_barrier()

        if with_gather:
            def _wait_scat(b):
                pltpu.make_async_copy(rows.at[b], acc.at[didx.at[0]],
                                      ssem[b]).wait()

            def _step(c, b, first):
                nb = 1 - b

                @pl.when(c + 1 < n_ch)
                def _():
                    _wait_idx(isem[nb])
                    if not first:
                        _wait_scat(nb)
                    pltpu.async_copy(x_hbm.at[sidx.at[nb]], rows.at[nb],
                                     gsem[nb])
                _wait_gather(b)
                pltpu.async_copy(rows.at[b], acc.at[didx.at[c]], ssem[b],
                                 add=True)

                @pl.when(c + 2 < n_ch)
                def _():
                    pltpu.async_copy(
                        src_hbm.at[pl.ds(base + (c + 2) * ch, ch)],
                        sidx.at[b], isem[b])

            _step(0, 0, True)

            def _grp(t, _):
                for b in range(NB):
                    _step(1 + t * NB + b, 1 - b if NB == 2 else b, False)
                return 0
            lax.fori_loop(0, (n_ch - 1) // NB, _grp, 0)
            _wait_scat((n_ch - 2) % NB)
            _wait_scat((n_ch - 1) % NB)
        else:
            dsems = list(sems)
            nd = len(dsems)
            for b in range(nd):
                pltpu.async_copy(rows.at[0], acc.at[didx.at[b]], dsems[b],
                                 add=True)

            def _grp(t, _):
                for b in range(nd):
                    c = (t + 1) * nd + b
                    pltpu.make_async_copy(rows.at[0], acc.at[didx.at[0]],
                                          dsems[b]).wait()

                    @pl.when(c < n_ch)
                    def _():
                        pltpu.async_copy(rows.at[0], acc.at[didx.at[c]],
                                         dsems[b], add=True)
                return 0
            lax.fori_loop(0, (n_ch + nd - 1) // nd - 1, _grp, 0)
            rem = n_ch - ((n_ch + nd - 1) // nd - 1) * nd
            for b in range(rem):
                pltpu.make_async_copy(rows.at[0], acc.at[didx.at[0]],
                                      dsems[b]).wait()
        plsc.subcore_barrier()

        out0 = pl.multiple_of(cid * n + row0, 8)
        pltpu.sync_copy(acc.at[pl.ds(row0, rpt)], out_hbm.at[pl.ds(out0, rpt)])

        @pl.when(sid == _NS - 1)
        def _():
            pltpu.sync_copy(acc.at[pl.ds(_NS * rpt, last_extra)],
                            out_hbm.at[pl.ds(cid * n + _NS * rpt, last_extra)])

    return seg


def _cast_kernel(n, d, br):
    def body(x_ref, o_ref):
        o_ref[...] = x_ref[...].astype(jnp.bfloat16)

    return pl.pallas_call(
        body,
        grid=(n // br,),
        in_specs=[pl.BlockSpec((br, d), lambda i: (i, 0))],
        out_specs=pl.BlockSpec((br, d), lambda i: (i, 0)),
        out_shape=jax.ShapeDtypeStruct((n, d), jnp.bfloat16),
    )


def _dense_layer(n, d, h, br):
    grid = (n // br,)

    def body(x_ref, p_ref, dg_ref, ws_ref, wn_ref, b_ref, o_ref, ob_ref):
        p = p_ref[0].astype(jnp.float32) + p_ref[1].astype(jnp.float32)
        deg = dg_ref[0] + dg_ref[1]
        hn = p / jnp.maximum(deg, 1.0)
        y = (jnp.dot(x_ref[...], ws_ref[...], preferred_element_type=jnp.float32)
             + jnp.dot(hn, wn_ref[...], preferred_element_type=jnp.float32)
             + b_ref[...])
        y = jnp.maximum(y, 0.0)
        o_ref[...] = y
        ob_ref[...] = y.astype(jnp.bfloat16)

    return pl.pallas_call(
        body,
        grid=grid,
        in_specs=[
            pl.BlockSpec((br, d), lambda i: (i, 0)),
            pl.BlockSpec((_NC, br, d), lambda i: (0, i, 0)),
            pl.BlockSpec((_NC, br, d), lambda i: (0, i, 0)),
            pl.BlockSpec((d, h), lambda i: (0, 0)),
            pl.BlockSpec((d, h), lambda i: (0, 0)),
            pl.BlockSpec((1, h), lambda i: (0, 0)),
        ],
        out_specs=[pl.BlockSpec((br, h), lambda i: (i, 0)),
                   pl.BlockSpec((br, h), lambda i: (i, 0))],
        out_shape=[jax.ShapeDtypeStruct((n, h), jnp.float32),
                   jax.ShapeDtypeStruct((n, h), jnp.bfloat16)],
    )


def _final_layer(n, d, h, fc2, c, br):
    nb = n // br

    def body(x_ref, p_ref, dg_ref, ws_ref, wn_ref, b_ref,
             wf1_ref, bf1_ref, wf2_ref, bf2_ref, o_ref, acc_ref):
        i = pl.program_id(0)

        @pl.when(i == 0)
        def _():
            acc_ref[...] = jnp.zeros_like(acc_ref)

        p = p_ref[0].astype(jnp.float32) + p_ref[1].astype(jnp.float32)
        deg = dg_ref[0] + dg_ref[1]
        hn = p / jnp.maximum(deg, 1.0)
        y = (jnp.dot(x_ref[...], ws_ref[...], preferred_element_type=jnp.float32)
             + jnp.dot(hn, wn_ref[...], preferred_element_type=jnp.float32)
             + b_ref[...])
        y = jnp.maximum(y, 0.0)
        acc_ref[...] += jnp.sum(y, axis=0, keepdims=True)

        @pl.when(i == nb - 1)
        def _():
            hg = acc_ref[...] / float(n)
            t1 = jnp.dot(hg, wf1_ref[...], preferred_element_type=jnp.float32)
            t1 = jnp.maximum(t1 + bf1_ref[...], 0.0)
            t2 = jnp.dot(t1, wf2_ref[...], preferred_element_type=jnp.float32)
            t2 = t2 + bf2_ref[...]
            m = jnp.max(t2)
            lse = m + jnp.log(jnp.sum(jnp.exp(t2 - m)))
            o_ref[...] = t2 - lse

    return pl.pallas_call(
        body,
        grid=(nb,),
        in_specs=[
            pl.BlockSpec((br, d), lambda i: (i, 0)),
            pl.BlockSpec((_NC, br, d), lambda i: (0, i, 0)),
            pl.BlockSpec((_NC, br, d), lambda i: (0, i, 0)),
            pl.BlockSpec((d, h), lambda i: (0, 0)),
            pl.BlockSpec((d, h), lambda i: (0, 0)),
            pl.BlockSpec((1, h), lambda i: (0, 0)),
            pl.BlockSpec((h, fc2), lambda i: (0, 0)),
            pl.BlockSpec((1, fc2), lambda i: (0, 0)),
            pl.BlockSpec((fc2, c), lambda i: (0, 0)),
            pl.BlockSpec((1, c), lambda i: (0, 0)),
        ],
        out_specs=pl.BlockSpec((1, c), lambda i: (0, 0)),
        out_shape=jax.ShapeDtypeStruct((1, c), jnp.float32),
        scratch_shapes=[pltpu.VMEM((1, h), jnp.float32)],
    )


def kernel(x, edge_index, W1_self, W1_neigh, b1, W2_self, W2_neigh, b2,
           W3_self, W3_neigh, b3, Wfc1, bfc1, Wfc2, bfc2):
    n, d = x.shape
    e = edge_index.shape[1]
    h1 = W1_self.shape[1]
    h2 = W2_self.shape[1]
    fc1 = W3_self.shape[1]
    fc2 = Wfc1.shape[1]
    c = Wfc2.shape[1]
    br = 2000

    src1 = edge_index[0]
    dst1 = edge_index[1]

    seg = _seg_sum_kernel(n, d, e, with_gather=True, dtype=jnp.bfloat16)
    degk = _seg_sum_kernel(n, d, e, with_gather=False, dtype=jnp.float32)
    cast = _cast_kernel(n, d, br)
    dense1 = _dense_layer(n, d, h1, br)
    dense2 = _dense_layer(n, h1, h2, br)
    dense3 = _final_layer(n, h2, fc1, fc2, c, br)

    degw = degk(x, src1, dst1).reshape(_NC, n, d)

    xb = cast(x)
    p1 = seg(xb, src1, dst1).reshape(_NC, n, d)
    hh1, hb1 = dense1(x, p1, degw, W1_self, W1_neigh, b1.reshape(1, h1))
    p2 = seg(hb1, src1, dst1).reshape(_NC, n, d)
    hh2, hb2 = dense2(hh1, p2, degw, W2_self, W2_neigh, b2.reshape(1, h2))
    p3 = seg(hb2, src1, dst1).reshape(_NC, n, d)
    out = dense3(hh2, p3, degw, W3_self, W3_neigh, b3.reshape(1, fc1),
                 Wfc1, bfc1.reshape(1, fc2), Wfc2, bfc2.reshape(1, c))
    return out


# speedup vs baseline: 1.4050x; 1.4050x over previous
import functools

import jax
import jax.numpy as jnp
from jax import lax
from jax.experimental import pallas as pl
from jax.experimental.pallas import tpu as pltpu
from jax.experimental.pallas import tpu_sc as plsc

_NC, _NS, _L = 2, 16, 16
_NW = _NC * _NS


def _seg_sum_kernel(n, d, e, with_gather, dtype=jnp.float32, ch=80):
    per_w = e // _NW
    n_ch = per_w // ch
    NB = 2
    n_grp = n_ch // NB
    n_tail = n_ch - n_grp * NB
    rpt = (n // _NS) // 8 * 8
    last_extra = n - _NS * rpt
    nz = rpt // ch
    zrem = rpt - nz * ch
    mesh = plsc.VectorSubcoreMesh(core_axis_name="c", subcore_axis_name="s")

    scratch = [
        pltpu.VMEM((NB, ch), jnp.int32),
        pltpu.VMEM((n_ch, ch), jnp.int32),
        pltpu.VMEM((NB, ch, d), dtype),
        pltpu.VMEM_SHARED((n, d), dtype),
    ] + [pltpu.SemaphoreType.DMA] * 8

    @functools.partial(
        pl.kernel,
        out_type=jax.ShapeDtypeStruct((_NC * n, d), dtype),
        mesh=mesh,
        scratch_types=scratch,
        compiler_params=pltpu.CompilerParams(use_tc_tiling_on_sc=False),
    )
    def seg(x_hbm, src_hbm, dst_hbm, out_hbm, sidx, didx, rows, acc,
            *sems):
        gsem, isem, ssem = sems[:NB], sems[NB:2 * NB], sems[2 * NB:3 * NB]
        cid = lax.axis_index("c")
        sid = lax.axis_index("s")
        wid = cid * _NS + sid
        base = wid * per_w

        def _wait_idx(sem):
            pltpu.make_async_copy(dst_hbm.at[pl.ds(0, ch)], sidx.at[0],
                                  sem).wait()

        def _wait_gather(b):
            pltpu.make_async_copy(x_hbm.at[sidx.at[b]], rows.at[b],
                                  gsem[b]).wait()

        ni = len(sems)
        ip_tail = n_ch % ni

        def _idx(t, _):
            for b in range(ni):
                k = t * ni + b

                @pl.when(t > 0)
                def _():
                    _wait_idx(sems[b])
                pltpu.async_copy(dst_hbm.at[pl.ds(base + k * ch, ch)],
                                 didx.at[k], sems[b])
            return 0
        lax.fori_loop(0, n_ch // ni, _idx, 0)
        for r in range(ip_tail):
            k = (n_ch // ni) * ni + r
            _wait_idx(sems[r])
            pltpu.async_copy(dst_hbm.at[pl.ds(base + k * ch, ch)],
                             didx.at[k], sems[r])

        lanes = _L * 4 // jnp.dtype(dtype).itemsize

        def _zr(r, _):
            def _zc(c, _):
                rows[0, r, pl.ds(c * lanes, lanes)] = jnp.zeros((lanes,),
                                                                dtype)
                return 0
            return lax.fori_loop(0, d // lanes, _zc, 0)
        lax.fori_loop(0, ch, _zr, 0)

        for b in range(ni):
            _wait_idx(sems[b])

        row0 = pl.multiple_of(sid * rpt, 8)
        for j in range(nz):
            pltpu.sync_copy(rows.at[0], acc.at[pl.ds(row0 + j * ch, ch)])
        pltpu.sync_copy(rows.at[0, pl.ds(0, zrem)],
                        acc.at[pl.ds(row0 + nz * ch, zrem)])

        @pl.when(sid == _NS - 1)
        def _():
            pltpu.sync_copy(rows.at[0, pl.ds(0, last_extra)],
                            acc.at[pl.ds(_NS * rpt, last_extra)])

        if with_gather:
            for b in range(NB):
                pltpu.async_copy(src_hbm.at[pl.ds(base + b * ch, ch)],
                                 sidx.at[b], isem[b])
            _wait_idx(isem[0])
            pltpu.async_copy(x_hbm.at[sidx.at[0]], rows.at[0], gsem[0])
        else:
            def _or(r, _):
                def _oc(c, _):
                    rows[0, r, pl.ds(c * lanes, lanes)] = jnp.full(
                        (lanes,), 1.0, dtype)
                    return 0
                return lax.fori_loop(0, d // lanes, _oc, 0)
            lax.fori_loop(0, ch, _or, 0)
        plsc.subcore_barrier()

        if with_gather:
            def _wait_scat(b):
                pltpu.make_async_copy(rows.at[b], acc.at[didx.at[0]],
                                      ssem[b]).wait()

            def _step(c, b, first):
                nb = 1 - b

                @pl.when(c + 1 < n_ch)
                def _():
                    _wait_idx(isem[nb])
                    if not first:
                        _wait_scat(nb)
                    pltpu.async_copy(x_hbm.at[sidx.at[nb]], rows.at[nb],
                                     gsem[nb])
                _wait_gather(b)
                pltpu.async_copy(rows.at[b], acc.at[didx.at[c]], ssem[b],
                                 add=True)

                @pl.when(c + 2 < n_ch)
                def _():
                    pltpu.async_copy(
                        src_hbm.at[pl.ds(base + (c + 2) * ch, ch)],
                        sidx.at[b], isem[b])

            _step(0, 0, True)

            def _grp(t, _):
                for b in range(NB):
                    _step(1 + t * NB + b, 1 - b if NB == 2 else b, False)
                return 0
            lax.fori_loop(0, (n_ch - 1) // NB, _grp, 0)
            _wait_scat((n_ch - 2) % NB)
            _wait_scat((n_ch - 1) % NB)
        else:
            dsems = list(sems)
            nd = len(dsems)
            for b in range(nd):
                pltpu.async_copy(rows.at[0], acc.at[didx.at[b]], dsems[b],
                                 add=True)

            def _grp(t, _):
                for b in range(nd):
                    c = (t + 1) * nd + b
                    pltpu.make_async_copy(rows.at[0], acc.at[didx.at[0]],
                                          dsems[b]).wait()

                    @pl.when(c < n_ch)
                    def _():
                        pltpu.async_copy(rows.at[0], acc.at[didx.at[c]],
                                         dsems[b], add=True)
                return 0
            lax.fori_loop(0, (n_ch + nd - 1) // nd - 1, _grp, 0)
            rem = n_ch - ((n_ch + nd - 1) // nd - 1) * nd
            for b in range(rem):
                pltpu.make_async_copy(rows.at[0], acc.at[didx.at[0]],
                                      dsems[b]).wait()
        plsc.subcore_barrier()

        out0 = pl.multiple_of(cid * n + row0, 8)
        pltpu.sync_copy(acc.at[pl.ds(row0, rpt)], out_hbm.at[pl.ds(out0, rpt)])

        @pl.when(sid == _NS - 1)
        def _():
            pltpu.sync_copy(acc.at[pl.ds(_NS * rpt, last_extra)],
                            out_hbm.at[pl.ds(cid * n + _NS * rpt, last_extra)])

    return seg


def _cast_kernel(n, d, br):
    def body(x_ref, o_ref):
        o_ref[...] = x_ref[...].astype(jnp.bfloat16)

    return pl.pallas_call(
        body,
        grid=(n // br,),
        in_specs=[pl.BlockSpec((br, d), lambda i: (i, 0))],
        out_specs=pl.BlockSpec((br, d), lambda i: (i, 0)),
        out_shape=jax.ShapeDtypeStruct((n, d), jnp.bfloat16),
    )


def _dense_layer(n, d, h, br):
    grid = (n // br,)

    def body(x_ref, p_ref, dg_ref, ws_ref, wn_ref, b_ref, o_ref, ob_ref):
        p = p_ref[0].astype(jnp.float32) + p_ref[1].astype(jnp.float32)
        deg = dg_ref[0].astype(jnp.float32) + dg_ref[1].astype(jnp.float32)
        hn = p / jnp.maximum(deg, 1.0)
        y = (jnp.dot(x_ref[...], ws_ref[...], preferred_element_type=jnp.float32)
             + jnp.dot(hn, wn_ref[...], preferred_element_type=jnp.float32)
             + b_ref[...])
        y = jnp.maximum(y, 0.0)
        o_ref[...] = y
        ob_ref[...] = y.astype(jnp.bfloat16)

    return pl.pallas_call(
        body,
        grid=grid,
        in_specs=[
            pl.BlockSpec((br, d), lambda i: (i, 0)),
            pl.BlockSpec((_NC, br, d), lambda i: (0, i, 0)),
            pl.BlockSpec((_NC, br, d), lambda i: (0, i, 0)),
            pl.BlockSpec((d, h), lambda i: (0, 0)),
            pl.BlockSpec((d, h), lambda i: (0, 0)),
            pl.BlockSpec((1, h), lambda i: (0, 0)),
        ],
        out_specs=[pl.BlockSpec((br, h), lambda i: (i, 0)),
                   pl.BlockSpec((br, h), lambda i: (i, 0))],
        out_shape=[jax.ShapeDtypeStruct((n, h), jnp.float32),
                   jax.ShapeDtypeStruct((n, h), jnp.bfloat16)],
    )


def _final_layer(n, d, h, fc2, c, br):
    nb = n // br

    def body(x_ref, p_ref, dg_ref, ws_ref, wn_ref, b_ref,
             wf1_ref, bf1_ref, wf2_ref, bf2_ref, o_ref, acc_ref):
        i = pl.program_id(0)

        @pl.when(i == 0)
        def _():
            acc_ref[...] = jnp.zeros_like(acc_ref)

        p = p_ref[0].astype(jnp.float32) + p_ref[1].astype(jnp.float32)
        deg = dg_ref[0].astype(jnp.float32) + dg_ref[1].astype(jnp.float32)
        hn = p / jnp.maximum(deg, 1.0)
        y = (jnp.dot(x_ref[...], ws_ref[...], preferred_element_type=jnp.float32)
             + jnp.dot(hn, wn_ref[...], preferred_element_type=jnp.float32)
             + b_ref[...])
        y = jnp.maximum(y, 0.0)
        acc_ref[...] += jnp.sum(y, axis=0, keepdims=True)

        @pl.when(i == nb - 1)
        def _():
            hg = acc_ref[...] / float(n)
            t1 = jnp.dot(hg, wf1_ref[...], preferred_element_type=jnp.float32)
            t1 = jnp.maximum(t1 + bf1_ref[...], 0.0)
            t2 = jnp.dot(t1, wf2_ref[...], preferred_element_type=jnp.float32)
            t2 = t2 + bf2_ref[...]
            m = jnp.max(t2)
            lse = m + jnp.log(jnp.sum(jnp.exp(t2 - m)))
            o_ref[...] = t2 - lse

    return pl.pallas_call(
        body,
        grid=(nb,),
        in_specs=[
            pl.BlockSpec((br, d), lambda i: (i, 0)),
            pl.BlockSpec((_NC, br, d), lambda i: (0, i, 0)),
            pl.BlockSpec((_NC, br, d), lambda i: (0, i, 0)),
            pl.BlockSpec((d, h), lambda i: (0, 0)),
            pl.BlockSpec((d, h), lambda i: (0, 0)),
            pl.BlockSpec((1, h), lambda i: (0, 0)),
            pl.BlockSpec((h, fc2), lambda i: (0, 0)),
            pl.BlockSpec((1, fc2), lambda i: (0, 0)),
            pl.BlockSpec((fc2, c), lambda i: (0, 0)),
            pl.BlockSpec((1, c), lambda i: (0, 0)),
        ],
        out_specs=pl.BlockSpec((1, c), lambda i: (0, 0)),
        out_shape=jax.ShapeDtypeStruct((1, c), jnp.float32),
        scratch_shapes=[pltpu.VMEM((1, h), jnp.float32)],
    )


def kernel(x, edge_index, W1_self, W1_neigh, b1, W2_self, W2_neigh, b2,
           W3_self, W3_neigh, b3, Wfc1, bfc1, Wfc2, bfc2):
    n, d = x.shape
    e = edge_index.shape[1]
    h1 = W1_self.shape[1]
    h2 = W2_self.shape[1]
    fc1 = W3_self.shape[1]
    fc2 = Wfc1.shape[1]
    c = Wfc2.shape[1]
    br = 2000

    src1 = edge_index[0]
    dst1 = edge_index[1]

    seg = _seg_sum_kernel(n, d, e, with_gather=True, dtype=jnp.bfloat16,
                          ch=400)
    degk = _seg_sum_kernel(n, d, e, with_gather=False, dtype=jnp.bfloat16,
                           ch=400)
    cast = _cast_kernel(n, d, br)
    dense1 = _dense_layer(n, d, h1, br)
    dense2 = _dense_layer(n, h1, h2, br)
    dense3 = _final_layer(n, h2, fc1, fc2, c, br)

    degw = degk(x, src1, dst1).reshape(_NC, n, d)

    xb = cast(x)
    p1 = seg(xb, src1, dst1).reshape(_NC, n, d)
    hh1, hb1 = dense1(x, p1, degw, W1_self, W1_neigh, b1.reshape(1, h1))
    p2 = seg(hb1, src1, dst1).reshape(_NC, n, d)
    hh2, hb2 = dense2(hh1, p2, degw, W2_self, W2_neigh, b2.reshape(1, h2))
    p3 = seg(hb2, src1, dst1).reshape(_NC, n, d)
    out = dense3(hh2, p3, degw, W3_self, W3_neigh, b3.reshape(1, fc1),
                 Wfc1, bfc1.reshape(1, fc2), Wfc2, bfc2.reshape(1, c))
    return out


# speedup vs baseline: 1.4543x; 1.0351x over previous
import functools

import jax
import jax.numpy as jnp
from jax import lax
from jax.experimental import pallas as pl
from jax.experimental.pallas import tpu as pltpu
from jax.experimental.pallas import tpu_sc as plsc

_NC, _NS, _L = 2, 16, 16
_NW = _NC * _NS


def _seg_sum_kernel(n, d, e, with_gather, dtype=jnp.float32, ch=80):
    per_w = e // _NW
    n_ch = per_w // ch
    NB = 2
    n_grp = n_ch // NB
    n_tail = n_ch - n_grp * NB
    rpt = (n // _NS) // 8 * 8
    last_extra = n - _NS * rpt
    nz = rpt // ch
    zrem = rpt - nz * ch
    mesh = plsc.VectorSubcoreMesh(core_axis_name="c", subcore_axis_name="s")

    scratch = [
        pltpu.VMEM((NB, ch), jnp.int32),
        pltpu.VMEM((n_ch, ch), jnp.int32),
        pltpu.VMEM((NB, ch, d), dtype),
        pltpu.VMEM_SHARED((n, d), dtype),
    ] + [pltpu.SemaphoreType.DMA] * 8

    @functools.partial(
        pl.kernel,
        out_type=jax.ShapeDtypeStruct((_NC * n, d), dtype),
        mesh=mesh,
        scratch_types=scratch,
        compiler_params=pltpu.CompilerParams(use_tc_tiling_on_sc=False),
    )
    def seg(x_hbm, src_hbm, dst_hbm, out_hbm, sidx, didx, rows, acc,
            *sems):
        gsem, isem, ssem = sems[:NB], sems[NB:2 * NB], sems[2 * NB:3 * NB]
        cid = lax.axis_index("c")
        sid = lax.axis_index("s")
        wid = cid * _NS + sid
        base = wid * per_w

        def _wait_idx(sem):
            pltpu.make_async_copy(dst_hbm.at[pl.ds(0, ch)], sidx.at[0],
                                  sem).wait()

        def _wait_gather(b):
            pltpu.make_async_copy(x_hbm.at[sidx.at[b]], rows.at[b],
                                  gsem[b]).wait()

        ni = min(len(sems), n_ch)
        ip_tail = n_ch % ni

        def _idx(t, _):
            for b in range(ni):
                k = t * ni + b

                @pl.when(t > 0)
                def _():
                    _wait_idx(sems[b])
                pltpu.async_copy(dst_hbm.at[pl.ds(base + k * ch, ch)],
                                 didx.at[k], sems[b])
            return 0
        lax.fori_loop(0, n_ch // ni, _idx, 0)
        for r in range(ip_tail):
            k = (n_ch // ni) * ni + r
            _wait_idx(sems[r])
            pltpu.async_copy(dst_hbm.at[pl.ds(base + k * ch, ch)],
                             didx.at[k], sems[r])

        lanes = _L * 4 // jnp.dtype(dtype).itemsize

        def _zr(r, _):
            def _zc(c, _):
                rows[0, r, pl.ds(c * lanes, lanes)] = jnp.zeros((lanes,),
                                                                dtype)
                return 0
            return lax.fori_loop(0, d // lanes, _zc, 0)
        lax.fori_loop(0, ch, _zr, 0)

        for b in range(min(ni, n_ch)):
            _wait_idx(sems[b])

        row0 = pl.multiple_of(sid * rpt, 8)
        for j in range(nz):
            pltpu.sync_copy(rows.at[0], acc.at[pl.ds(row0 + j * ch, ch)])
        pltpu.sync_copy(rows.at[0, pl.ds(0, zrem)],
                        acc.at[pl.ds(row0 + nz * ch, zrem)])

        @pl.when(sid == _NS - 1)
        def _():
            pltpu.sync_copy(rows.at[0, pl.ds(0, last_extra)],
                            acc.at[pl.ds(_NS * rpt, last_extra)])

        if with_gather:
            for b in range(NB):
                pltpu.async_copy(src_hbm.at[pl.ds(base + b * ch, ch)],
                                 sidx.at[b], isem[b])
            _wait_idx(isem[0])
            pltpu.async_copy(x_hbm.at[sidx.at[0]], rows.at[0], gsem[0])
        else:
            def _or(r, _):
                def _oc(c, _):
                    rows[0, r, pl.ds(c * lanes, lanes)] = jnp.full(
                        (lanes,), 1.0, dtype)
                    return 0
                return lax.fori_loop(0, d // lanes, _oc, 0)
            lax.fori_loop(0, ch, _or, 0)
        plsc.subcore_barrier()

        if with_gather:
            def _wait_scat(b):
                pltpu.make_async_copy(rows.at[b], acc.at[didx.at[0]],
                                      ssem[b]).wait()

            def _step(c, b, first):
                nb = 1 - b

                @pl.when(c + 1 < n_ch)
                def _():
                    _wait_idx(isem[nb])
                    if not first:
                        _wait_scat(nb)
                    pltpu.async_copy(x_hbm.at[sidx.at[nb]], rows.at[nb],
                                     gsem[nb])
                _wait_gather(b)
                pltpu.async_copy(rows.at[b], acc.at[didx.at[c]], ssem[b],
                                 add=True)

                @pl.when(c + 2 < n_ch)
                def _():
                    pltpu.async_copy(
                        src_hbm.at[pl.ds(base + (c + 2) * ch, ch)],
                        sidx.at[b], isem[b])

            _step(0, 0, True)

            def _grp(t, _):
                for b in range(NB):
                    _step(1 + t * NB + b, 1 - b if NB == 2 else b, False)
                return 0
            lax.fori_loop(0, (n_ch - 1) // NB, _grp, 0)
            _wait_scat((n_ch - 2) % NB)
            _wait_scat((n_ch - 1) % NB)
        else:
            dsems = list(sems)[:min(len(sems), n_ch)]
            nd = len(dsems)
            for b in range(nd):
                pltpu.async_copy(rows.at[0], acc.at[didx.at[b]], dsems[b],
                                 add=True)

            def _grp(t, _):
                for b in range(nd):
                    c = (t + 1) * nd + b
                    pltpu.make_async_copy(rows.at[0], acc.at[didx.at[0]],
                                          dsems[b]).wait()

                    @pl.when(c < n_ch)
                    def _():
                        pltpu.async_copy(rows.at[0], acc.at[didx.at[c]],
                                         dsems[b], add=True)
                return 0
            lax.fori_loop(0, (n_ch + nd - 1) // nd - 1, _grp, 0)
            rem = n_ch - ((n_ch + nd - 1) // nd - 1) * nd
            for b in range(rem):
                pltpu.make_async_copy(rows.at[0], acc.at[didx.at[0]],
                                      dsems[b]).wait()
        plsc.subcore_barrier()

        out0 = pl.multiple_of(cid * n + row0, 8)
        pltpu.sync_copy(acc.at[pl.ds(row0, rpt)], out_hbm.at[pl.ds(out0, rpt)])

        @pl.when(sid == _NS - 1)
        def _():
            pltpu.sync_copy(acc.at[pl.ds(_NS * rpt, last_extra)],
                            out_hbm.at[pl.ds(cid * n + _NS * rpt, last_extra)])

    return seg


def _cast_kernel(n, d, br):
    def body(x_ref, o_ref):
        o_ref[...] = x_ref[...].astype(jnp.bfloat16)

    return pl.pallas_call(
        body,
        grid=(n // br,),
        in_specs=[pl.BlockSpec((br, d), lambda i: (i, 0))],
        out_specs=pl.BlockSpec((br, d), lambda i: (i, 0)),
        out_shape=jax.ShapeDtypeStruct((n, d), jnp.bfloat16),
    )


def _dense_layer(n, d, h, br):
    grid = (n // br,)

    def body(x_ref, p_ref, dg_ref, ws_ref, wn_ref, b_ref, o_ref, ob_ref):
        p = p_ref[0].astype(jnp.float32) + p_ref[1].astype(jnp.float32)
        deg = (dg_ref[0, :, :1].astype(jnp.float32)
               + dg_ref[1, :, :1].astype(jnp.float32))
        hn = p / jnp.maximum(deg, 1.0)
        y = (jnp.dot(x_ref[...], ws_ref[...], preferred_element_type=jnp.float32)
             + jnp.dot(hn, wn_ref[...], preferred_element_type=jnp.float32)
             + b_ref[...])
        y = jnp.maximum(y, 0.0)
        o_ref[...] = y
        ob_ref[...] = y.astype(jnp.bfloat16)

    return pl.pallas_call(
        body,
        grid=grid,
        in_specs=[
            pl.BlockSpec((br, d), lambda i: (i, 0)),
            pl.BlockSpec((_NC, br, d), lambda i: (0, i, 0)),
            pl.BlockSpec((_NC, br, 32), lambda i: (0, i, 0)),
            pl.BlockSpec((d, h), lambda i: (0, 0)),
            pl.BlockSpec((d, h), lambda i: (0, 0)),
            pl.BlockSpec((1, h), lambda i: (0, 0)),
        ],
        out_specs=[pl.BlockSpec((br, h), lambda i: (i, 0)),
                   pl.BlockSpec((br, h), lambda i: (i, 0))],
        out_shape=[jax.ShapeDtypeStruct((n, h), jnp.float32),
                   jax.ShapeDtypeStruct((n, h), jnp.bfloat16)],
    )


def _final_layer(n, d, h, fc2, c, br):
    nb = n // br

    def body(x_ref, p_ref, dg_ref, ws_ref, wn_ref, b_ref,
             wf1_ref, bf1_ref, wf2_ref, bf2_ref, o_ref, acc_ref):
        i = pl.program_id(0)

        @pl.when(i == 0)
        def _():
            acc_ref[...] = jnp.zeros_like(acc_ref)

        p = p_ref[0].astype(jnp.float32) + p_ref[1].astype(jnp.float32)
        deg = (dg_ref[0, :, :1].astype(jnp.float32)
               + dg_ref[1, :, :1].astype(jnp.float32))
        hn = p / jnp.maximum(deg, 1.0)
        y = (jnp.dot(x_ref[...], ws_ref[...], preferred_element_type=jnp.float32)
             + jnp.dot(hn, wn_ref[...], preferred_element_type=jnp.float32)
             + b_ref[...])
        y = jnp.maximum(y, 0.0)
        acc_ref[...] += jnp.sum(y, axis=0, keepdims=True)

        @pl.when(i == nb - 1)
        def _():
            hg = acc_ref[...] / float(n)
            t1 = jnp.dot(hg, wf1_ref[...], preferred_element_type=jnp.float32)
            t1 = jnp.maximum(t1 + bf1_ref[...], 0.0)
            t2 = jnp.dot(t1, wf2_ref[...], preferred_element_type=jnp.float32)
            t2 = t2 + bf2_ref[...]
            m = jnp.max(t2)
            lse = m + jnp.log(jnp.sum(jnp.exp(t2 - m)))
            o_ref[...] = t2 - lse

    return pl.pallas_call(
        body,
        grid=(nb,),
        in_specs=[
            pl.BlockSpec((br, d), lambda i: (i, 0)),
            pl.BlockSpec((_NC, br, d), lambda i: (0, i, 0)),
            pl.BlockSpec((_NC, br, 32), lambda i: (0, i, 0)),
            pl.BlockSpec((d, h), lambda i: (0, 0)),
            pl.BlockSpec((d, h), lambda i: (0, 0)),
            pl.BlockSpec((1, h), lambda i: (0, 0)),
            pl.BlockSpec((h, fc2), lambda i: (0, 0)),
            pl.BlockSpec((1, fc2), lambda i: (0, 0)),
            pl.BlockSpec((fc2, c), lambda i: (0, 0)),
            pl.BlockSpec((1, c), lambda i: (0, 0)),
        ],
        out_specs=pl.BlockSpec((1, c), lambda i: (0, 0)),
        out_shape=jax.ShapeDtypeStruct((1, c), jnp.float32),
        scratch_shapes=[pltpu.VMEM((1, h), jnp.float32)],
    )


def kernel(x, edge_index, W1_self, W1_neigh, b1, W2_self, W2_neigh, b2,
           W3_self, W3_neigh, b3, Wfc1, bfc1, Wfc2, bfc2):
    n, d = x.shape
    e = edge_index.shape[1]
    h1 = W1_self.shape[1]
    h2 = W2_self.shape[1]
    fc1 = W3_self.shape[1]
    fc2 = Wfc1.shape[1]
    c = Wfc2.shape[1]
    br = 2000

    src1 = edge_index[0]
    dst1 = edge_index[1]

    seg = _seg_sum_kernel(n, d, e, with_gather=True, dtype=jnp.bfloat16,
                          ch=400)
    degk = _seg_sum_kernel(n, 32, e, with_gather=False, dtype=jnp.bfloat16,
                           ch=2000)
    cast = _cast_kernel(n, d, br)
    dense1 = _dense_layer(n, d, h1, br)
    dense2 = _dense_layer(n, h1, h2, br)
    dense3 = _final_layer(n, h2, fc1, fc2, c, br)

    degw = degk(x, src1, dst1).reshape(_NC, n, 32)

    xb = cast(x)
    p1 = seg(xb, src1, dst1).reshape(_NC, n, d)
    hh1, hb1 = dense1(x, p1, degw, W1_self, W1_neigh, b1.reshape(1, h1))
    p2 = seg(hb1, src1, dst1).reshape(_NC, n, d)
    hh2, hb2 = dense2(hh1, p2, degw, W2_self, W2_neigh, b2.reshape(1, h2))
    p3 = seg(hb2, src1, dst1).reshape(_NC, n, d)
    out = dense3(hh2, p3, degw, W3_self, W3_neigh, b3.reshape(1, fc1),
                 Wfc1, bfc1.reshape(1, fc2), Wfc2, bfc2.reshape(1, c))
    return out
